# baseline (speedup 1.0000x reference)
import functools

import jax
import jax.numpy as jnp
from jax import lax
from jax.experimental import pallas as pl
from jax.experimental.pallas import tpu as pltpu
from jax.experimental.pallas import tpu_sc as plsc

VOCAB = 1000000
EMBED = 64
BATCH = 4096
SEQ = 200
SCALE = 8.0

NC = 2
NS = 16
NW = NC * NS

B_TOTAL = BATCH * SEQ
B_PER_W = B_TOTAL // NW
CHUNK = 640
N_CHUNKS = B_PER_W // CHUNK

_mesh = plsc.VectorSubcoreMesh(core_axis_name="c", subcore_axis_name="s")


@functools.partial(
    pl.kernel,
    mesh=_mesh,
    out_type=jax.ShapeDtypeStruct((B_TOTAL, EMBED), jnp.float32),
    scratch_types=[
        pltpu.VMEM((CHUNK,), jnp.int32),
        pltpu.VMEM((CHUNK, EMBED), jnp.float32),
        pltpu.SemaphoreType.DMA,
    ],
    compiler_params=pltpu.CompilerParams(use_tc_tiling_on_sc=False),
)
def _emb_lookup(idx_hbm, table_hbm, out_hbm, idx_v, rows_v, sem):
    wid = lax.axis_index("s") * NC + lax.axis_index("c")
    base = wid * B_PER_W

    def chunk_body(j, carry):
        cbase = base + j * CHUNK
        pltpu.sync_copy(idx_hbm.at[pl.ds(cbase, CHUNK)], idx_v)
        pltpu.async_copy(table_hbm.at[idx_v], rows_v, sem).wait()
        def row_body(r, c2):
            for c in range(EMBED // 16):
                sl = pl.ds(c * 16, 16)
                rows_v[r, sl] = rows_v[r, sl] * SCALE
            return c2
        lax.fori_loop(0, CHUNK, row_body, 0, unroll=2)
        pltpu.sync_copy(rows_v, out_hbm.at[pl.ds(cbase, CHUNK)])
        return carry

    lax.fori_loop(0, N_CHUNKS, chunk_body, 0)


def kernel(input_ids, table):
    idx = input_ids.reshape(B_TOTAL).astype(jnp.int32)
    out = _emb_lookup(idx, table)
    return out.reshape(BATCH, SEQ, EMBED)
